# baseline (speedup 1.0000x reference)
import functools

import jax
import jax.numpy as jnp
from jax import lax
from jax.experimental import pallas as pl
from jax.experimental.pallas import tpu as pltpu
from jax.experimental.pallas import tpu_sc as plsc

GRP = 2
NB = 8


@functools.lru_cache(maxsize=None)
def _build(n_rows: int, seq: int, vocab: int, d: int):
    info = plsc.get_sparse_core_info()
    nc, ns = info.num_cores, info.num_subcores
    nw = nc * ns
    rows_per_w = n_rows // nw
    units_per_w = rows_per_w // GRP
    u_idx = GRP * seq
    assert n_rows % nw == 0 and rows_per_w % (GRP * NB) == 0
    assert u_idx <= 128

    mesh = plsc.VectorSubcoreMesh(
        core_axis_name="c", subcore_axis_name="s",
        num_cores=nc, num_subcores=ns,
    )

    @functools.partial(
        pl.kernel,
        out_type=jax.ShapeDtypeStruct((n_rows, seq, d), jnp.float32),
        mesh=mesh,
        scratch_types=[
            pltpu.VMEM((units_per_w, u_idx), jnp.int32),
            pltpu.VMEM((NB, u_idx, d), jnp.float32),
        ] + [pltpu.SemaphoreType.DMA] * (2 * NB),
        compiler_params=pltpu.CompilerParams(use_tc_tiling_on_sc=False),
    )
    def k(idx_hbm, table_hbm, out_hbm, idx_v, rows_v, *sems):
        gsem, ssem = sems[:NB], sems[NB:]
        wid = lax.axis_index("s") * nc + lax.axis_index("c")
        row0 = wid * rows_per_w
        unit0 = wid * units_per_w
        pltpu.sync_copy(idx_hbm.at[pl.ds(unit0, units_per_w)], idx_v)

        def body(it, _):
            u0 = it * NB
            gathers = []
            for b in range(NB):
                cp = pltpu.async_copy(
                    table_hbm.at[idx_v.at[u0 + b]], rows_v.at[b], gsem[b])
                gathers.append(cp)
            stores = []
            for b in range(NB):
                gathers[b].wait()
                for g in range(GRP):
                    sp = pltpu.async_copy(
                        rows_v.at[b].at[pl.ds(g * seq, seq)],
                        out_hbm.at[row0 + (u0 + b) * GRP + g],
                        ssem[b])
                    stores.append(sp)
            for sp in stores:
                sp.wait()
            return 0

        lax.fori_loop(0, units_per_w // NB, body, 0)

    return k


def kernel(token_ids, weight):
    n_rows, seq = token_ids.shape
    idx2d = token_ids.astype(jnp.int32).reshape(n_rows // GRP, GRP * seq)
    k = _build(n_rows, seq, weight.shape[0], weight.shape[1])
    return k(idx2d, weight)
